# Initial kernel scaffold; baseline (speedup 1.0000x reference)
import jax
import jax.numpy as jnp
from jax.experimental import pallas as pl


def kernel(mem, task_ids, idx, val, new_task_ids):
    raise NotImplementedError("write your pallas kernel here")


# speedup vs baseline: 1.4259x; 1.4259x over previous
import functools

import jax
import jax.numpy as jnp
from jax import lax
from jax.experimental import pallas as pl
from jax.experimental.pallas import tpu as pltpu
from jax.experimental.pallas import tpu_sc as plsc

NC = 2
NS = 16
L = 16


def _sc_store(mem_hbm, tid_hbm, idx_hbm, val_hbm, ntid_hbm,
              out_hbm, otid_hbm,
              idx_v, ntid_v, win_v, rows_v, jlist_v, tid_v, tmp_v, mbuf_v,
              sem):
  M, D = mem_hbm.shape
  B = idx_hbm.shape[0]
  NW = NC * NS
  R = L * ((M + L * NW - 1) // (L * NW))
  NV = R // L

  w = lax.axis_index("c") * NS + lax.axis_index("s")
  base = jnp.minimum(w * R, M - R)
  lane = lax.iota(jnp.int32, L)

  pltpu.sync_copy(idx_hbm, idx_v)
  pltpu.sync_copy(ntid_hbm, ntid_v)
  pltpu.sync_copy(tid_hbm.at[pl.ds(base, R)], tid_v)

  for i in range(NV):
    win_v[pl.ds(i * L, L)] = jnp.full((L,), -1, jnp.int32)
    rows_v[pl.ds(i * L, L)] = jnp.full((L,), -1, jnp.int32)
    jlist_v[pl.ds(i * L, L)] = jnp.full((L,), -1, jnp.int32)
  rows_v[pl.ds(NV * L, L)] = jnp.full((L,), -1, jnp.int32)
  jlist_v[pl.ds(NV * L, L)] = jnp.full((L,), -1, jnp.int32)

  def phase_a(c, carry):
    off = c * L
    iv = idx_v[pl.ds(off, L)]
    _, last = plsc.scan_count(iv)
    keep = last & (iv >= base) & (iv < base + R)
    loc = jnp.where(keep, iv - base, 0)
    plsc.store_scatter(win_v, [loc], off + lane, mask=keep)
    return carry

  lax.fori_loop(0, B // L, phase_a, 0)

  pltpu.sync_copy(mem_hbm.at[pl.ds(base, R)], mbuf_v)
  pltpu.sync_copy(mbuf_v, out_hbm.at[pl.ds(base, R)])
  for i in range(NV):
    wv = win_v[pl.ds(i * L, L)]
    have = wv >= 0
    nv = plsc.load_gather(ntid_v, [jnp.where(have, wv, 0)], mask=have)
    tid_v[pl.ds(i * L, L)] = jnp.where(have, nv, tid_v[pl.ds(i * L, L)])
  pltpu.sync_copy(tid_v, otid_hbm.at[pl.ds(base, R)])

  def phase_c(i, cnt):
    wv = win_v[pl.ds(i * L, L)]
    have = wv >= 0
    rowv = base + i * L + lane
    plsc.store_compressed(rows_v.at[pl.ds(cnt, L)], rowv, mask=have)
    plsc.store_compressed(jlist_v.at[pl.ds(cnt, L)], wv, mask=have)
    npc = plsc.all_reduce_population_count(have)
    return cnt + lax.reduce_max(npc, (0,))

  cnt = lax.fori_loop(0, NV, phase_c, 0)

  plsc.subcore_barrier()

  def phase_d(c, carry):
    jv = jlist_v[pl.ds(c * L, L)]
    rv = rows_v[pl.ds(c * L, L)]
    valid = (c * L + lane) < cnt
    comp = jnp.where(valid, rv * 8192 + jv, -1)
    pad = plsc.cummax(comp)
    jv = jnp.where(valid, jv, lax.bitwise_and(pad, 8191))
    rv = jnp.where(valid, rv, lax.shift_right_logical(pad, 13))
    pltpu.async_copy(val_hbm.at[jv], tmp_v, sem).wait()
    pltpu.async_copy(tmp_v, out_hbm.at[rv], sem).wait()
    return carry

  nchunks = (cnt + L - 1) // L
  lax.fori_loop(0, nchunks, phase_d, 0)


@jax.jit
def kernel(mem, task_ids, idx, val, new_task_ids):
  M, D = mem.shape
  B = idx.shape[0]
  NW = NC * NS
  R = L * ((M + L * NW - 1) // (L * NW))

  mesh = plsc.VectorSubcoreMesh(
      core_axis_name="c", subcore_axis_name="s", num_cores=NC,
      num_subcores=NS)
  f = pl.kernel(
      _sc_store,
      out_type=(
          jax.ShapeDtypeStruct((M, D), jnp.float32),
          jax.ShapeDtypeStruct((M,), jnp.int32),
      ),
      mesh=mesh,
      compiler_params=pltpu.CompilerParams(needs_layout_passes=False),
      scratch_types=[
          pltpu.VMEM((B,), jnp.int32),
          pltpu.VMEM((B,), jnp.int32),
          pltpu.VMEM((R,), jnp.int32),
          pltpu.VMEM((R + L,), jnp.int32),
          pltpu.VMEM((R + L,), jnp.int32),
          pltpu.VMEM((R,), jnp.int32),
          pltpu.VMEM((L, D), jnp.float32),
          pltpu.VMEM((R, D), jnp.float32),
          pltpu.SemaphoreType.DMA,
      ],
  )
  return f(mem, task_ids, idx, val, new_task_ids)


# speedup vs baseline: 1.7931x; 1.2575x over previous
import functools

import jax
import jax.numpy as jnp
from jax import lax
from jax.experimental import pallas as pl
from jax.experimental.pallas import tpu as pltpu
from jax.experimental.pallas import tpu_sc as plsc

NC = 2
NS = 16
L = 16
UNROLL = 4


def _sc_store(mem_hbm, tid_hbm, idx_hbm, val_hbm, ntid_hbm,
              out_hbm, otid_hbm,
              idx_v, ntid_v, win_v, rows_v, jlist_v, tid_v, mbuf_v,
              sem0, rsem, wsem, gsem, ssem):
  M, D = mem_hbm.shape
  B = idx_hbm.shape[0]
  NW = NC * NS
  R = L * ((M + L * NW - 1) // (L * NW))
  NV = R // L

  w = lax.axis_index("c") * NS + lax.axis_index("s")
  base = jnp.minimum(w * R, M - R)
  lane = lax.iota(jnp.int32, L)

  cp_idx = pltpu.async_copy(idx_hbm, idx_v, sem0)
  cp_nt = pltpu.async_copy(ntid_hbm, ntid_v, sem0)
  cp_tid = pltpu.async_copy(tid_hbm.at[pl.ds(base, R)], tid_v, sem0)
  cp_mem = pltpu.async_copy(mem_hbm.at[pl.ds(base, R)], mbuf_v, rsem)

  for i in range(NV):
    win_v[pl.ds(i * L, L)] = jnp.full((L,), -1, jnp.int32)

  cp_idx.wait()
  cp_nt.wait()
  cp_tid.wait()

  def phase_a(cc, carry):
    for u in range(UNROLL):
      c = cc * UNROLL + u
      off = c * L
      iv = idx_v[pl.ds(off, L)]
      _, last = plsc.scan_count(iv)
      keep = last & (iv >= base) & (iv < base + R)
      loc = jnp.where(keep, iv - base, 0)
      plsc.store_scatter(win_v, [loc], off + lane, mask=keep)
    return carry

  lax.fori_loop(0, B // (L * UNROLL), phase_a, 0)

  cp_mem.wait()
  cp_out = pltpu.async_copy(mbuf_v, out_hbm.at[pl.ds(base, R)], wsem)

  for i in range(NV):
    wv = win_v[pl.ds(i * L, L)]
    have = wv >= 0
    nv = plsc.load_gather(ntid_v, [jnp.where(have, wv, 0)], mask=have)
    tid_v[pl.ds(i * L, L)] = jnp.where(have, nv, tid_v[pl.ds(i * L, L)])
  pltpu.sync_copy(tid_v, otid_hbm.at[pl.ds(base, R)])

  def phase_c(i, cnt):
    wv = win_v[pl.ds(i * L, L)]
    have = wv >= 0
    rowv = base + i * L + lane
    plsc.store_compressed(rows_v.at[pl.ds(cnt, L)], rowv, mask=have)
    plsc.store_compressed(jlist_v.at[pl.ds(cnt, L)], wv, mask=have)
    npc = plsc.all_reduce_population_count(have)
    return cnt + lax.reduce_max(npc, (0,))

  cnt = lax.fori_loop(0, NV, phase_c, 0)
  nchunks = (cnt + L - 1) // L

  def pad_lists(c, carry):
    jv = jlist_v[pl.ds(c * L, L)]
    rv = rows_v[pl.ds(c * L, L)]
    valid = (c * L + lane) < cnt
    comp = jnp.where(valid, rv * 8192 + jv, -1)
    pad = plsc.cummax(comp)
    jlist_v[pl.ds(c * L, L)] = jnp.where(
        valid, jv, lax.bitwise_and(pad, 8191))
    rows_v[pl.ds(c * L, L)] = jnp.where(
        valid, rv, lax.shift_right_logical(pad, 13))
    return carry

  lax.fori_loop(jnp.maximum(nchunks - 1, 0), nchunks, pad_lists, 0)

  cp_out.wait()

  plsc.subcore_barrier()

  def fire_gather(c, carry):
    jv = jlist_v[pl.ds(c * L, L)]
    pltpu.async_copy(val_hbm.at[jv], mbuf_v.at[pl.ds(c * L, L)], gsem)
    return carry

  def drain_gather(c, carry):
    jv = jlist_v[pl.ds(c * L, L)]
    pltpu.make_async_copy(
        val_hbm.at[jv], mbuf_v.at[pl.ds(c * L, L)], gsem).wait()
    return carry

  def fire_scatter(c, carry):
    rv = rows_v[pl.ds(c * L, L)]
    pltpu.async_copy(mbuf_v.at[pl.ds(c * L, L)], out_hbm.at[rv], ssem)
    return carry

  def drain_scatter(c, carry):
    rv = rows_v[pl.ds(c * L, L)]
    pltpu.make_async_copy(
        mbuf_v.at[pl.ds(c * L, L)], out_hbm.at[rv], ssem).wait()
    return carry

  lax.fori_loop(0, nchunks, fire_gather, 0)
  lax.fori_loop(0, nchunks, drain_gather, 0)
  lax.fori_loop(0, nchunks, fire_scatter, 0)
  lax.fori_loop(0, nchunks, drain_scatter, 0)


@jax.jit
def kernel(mem, task_ids, idx, val, new_task_ids):
  M, D = mem.shape
  B = idx.shape[0]
  NW = NC * NS
  R = L * ((M + L * NW - 1) // (L * NW))

  mesh = plsc.VectorSubcoreMesh(
      core_axis_name="c", subcore_axis_name="s", num_cores=NC,
      num_subcores=NS)
  f = pl.kernel(
      _sc_store,
      out_type=(
          jax.ShapeDtypeStruct((M, D), jnp.float32),
          jax.ShapeDtypeStruct((M,), jnp.int32),
      ),
      mesh=mesh,
      compiler_params=pltpu.CompilerParams(needs_layout_passes=False),
      scratch_types=[
          pltpu.VMEM((B,), jnp.int32),
          pltpu.VMEM((B,), jnp.int32),
          pltpu.VMEM((R,), jnp.int32),
          pltpu.VMEM((R + L,), jnp.int32),
          pltpu.VMEM((R + L,), jnp.int32),
          pltpu.VMEM((R,), jnp.int32),
          pltpu.VMEM((R, D), jnp.float32),
          pltpu.SemaphoreType.DMA,
          pltpu.SemaphoreType.DMA,
          pltpu.SemaphoreType.DMA,
          pltpu.SemaphoreType.DMA,
          pltpu.SemaphoreType.DMA,
      ],
  )
  return f(mem, task_ids, idx, val, new_task_ids)


# speedup vs baseline: 2.0554x; 1.1463x over previous
import functools

import jax
import jax.numpy as jnp
from jax import lax
from jax.experimental import pallas as pl
from jax.experimental.pallas import tpu as pltpu
from jax.experimental.pallas import tpu_sc as plsc

NC = 2
NS = 16
L = 16
UNROLL = 4


def _sc_store(mem_hbm, tid_hbm, idx_hbm, val_hbm, ntid_hbm,
              out_hbm, otid_hbm,
              idx_v, ntid_v, win_v, rows_v, jlist_v, tid_v, mbuf_v,
              sem0, rsem, wsem, gsem, ssem):
  M, D = mem_hbm.shape
  B = idx_hbm.shape[0]
  NW = NC * NS
  R = L * ((M + L * NW - 1) // (L * NW))
  NV = R // L

  w = lax.axis_index("c") * NS + lax.axis_index("s")
  base = jnp.minimum(w * R, M - R)
  lane = lax.iota(jnp.int32, L)

  cp_idx = pltpu.async_copy(idx_hbm, idx_v, sem0)
  cp_nt = pltpu.async_copy(ntid_hbm, ntid_v, sem0)
  cp_tid = pltpu.async_copy(tid_hbm.at[pl.ds(base, R)], tid_v, sem0)
  cp_mem = pltpu.async_copy(mem_hbm.at[pl.ds(base, R)], mbuf_v, rsem)

  for i in range(NV):
    win_v[pl.ds(i * L, L)] = jnp.full((L,), -1, jnp.int32)

  cp_idx.wait()
  cp_nt.wait()
  cp_tid.wait()

  def phase_a(cc, carry):
    ivs, lasts = [], []
    for u in range(UNROLL):
      iv = idx_v[pl.ds((cc * UNROLL + u) * L, L)]
      ivs.append(iv)
    for u in range(UNROLL):
      _, last = plsc.scan_count(ivs[u])
      lasts.append(last)
    for u in range(UNROLL):
      iv = ivs[u]
      keep = lasts[u] & (iv >= base) & (iv < base + R)
      loc = jnp.where(keep, iv - base, 0)
      plsc.store_scatter(win_v, [loc], (cc * UNROLL + u) * L + lane,
                         mask=keep)
    return carry

  with jax.named_scope("phase_a"):
    lax.fori_loop(0, B // (L * UNROLL), phase_a, 0)

  with jax.named_scope("wait_mem_read"):
    cp_mem.wait()
  cp_out = pltpu.async_copy(mbuf_v, out_hbm.at[pl.ds(base, R)], wsem)

  with jax.named_scope("tid_resolve"):
    for i in range(NV):
      wv = win_v[pl.ds(i * L, L)]
      have = wv >= 0
      nv = plsc.load_gather(ntid_v, [jnp.where(have, wv, 0)], mask=have)
      tid_v[pl.ds(i * L, L)] = jnp.where(have, nv, tid_v[pl.ds(i * L, L)])
    pltpu.sync_copy(tid_v, otid_hbm.at[pl.ds(base, R)])

  def phase_c(i, cnt):
    wv = win_v[pl.ds(i * L, L)]
    have = wv >= 0
    rowv = base + i * L + lane
    plsc.store_compressed(rows_v.at[pl.ds(cnt, L)], rowv, mask=have)
    plsc.store_compressed(jlist_v.at[pl.ds(cnt, L)], wv, mask=have)
    npc = plsc.all_reduce_population_count(have)
    return cnt + lax.reduce_max(npc, (0,))

  with jax.named_scope("phase_c"):
    cnt = lax.fori_loop(0, NV, phase_c, 0)
  nchunks = (cnt + L - 1) // L

  def pad_lists(c, carry):
    jv = jlist_v[pl.ds(c * L, L)]
    rv = rows_v[pl.ds(c * L, L)]
    valid = (c * L + lane) < cnt
    comp = jnp.where(valid, rv * 8192 + jv, -1)
    pad = plsc.cummax(comp)
    jlist_v[pl.ds(c * L, L)] = jnp.where(
        valid, jv, lax.bitwise_and(pad, 8191))
    rows_v[pl.ds(c * L, L)] = jnp.where(
        valid, rv, lax.shift_right_logical(pad, 13))
    return carry

  lax.fori_loop(jnp.maximum(nchunks - 1, 0), nchunks, pad_lists, 0)

  with jax.named_scope("wait_mem_write"):
    cp_out.wait()

  with jax.named_scope("barrier"):
    plsc.subcore_barrier()

  def fire_gather(c, carry):
    jv = jlist_v[pl.ds(c * L, L)]
    pltpu.async_copy(val_hbm.at[jv], mbuf_v.at[pl.ds(c * L, L)], gsem)
    return carry

  def drain_gather(c, carry):
    jv = jlist_v[pl.ds(c * L, L)]
    pltpu.make_async_copy(
        val_hbm.at[jv], mbuf_v.at[pl.ds(c * L, L)], gsem).wait()
    return carry

  def fire_scatter(c, carry):
    rv = rows_v[pl.ds(c * L, L)]
    pltpu.async_copy(mbuf_v.at[pl.ds(c * L, L)], out_hbm.at[rv], ssem)
    return carry

  def drain_scatter(c, carry):
    rv = rows_v[pl.ds(c * L, L)]
    pltpu.make_async_copy(
        mbuf_v.at[pl.ds(c * L, L)], out_hbm.at[rv], ssem).wait()
    return carry

  with jax.named_scope("phase_d_gather"):
    lax.fori_loop(0, nchunks, fire_gather, 0)
    lax.fori_loop(0, nchunks, drain_gather, 0)
  with jax.named_scope("phase_d_scatter"):
    lax.fori_loop(0, nchunks, fire_scatter, 0)
    lax.fori_loop(0, nchunks, drain_scatter, 0)


@jax.jit
def kernel(mem, task_ids, idx, val, new_task_ids):
  M, D = mem.shape
  B = idx.shape[0]
  NW = NC * NS
  R = L * ((M + L * NW - 1) // (L * NW))

  mesh = plsc.VectorSubcoreMesh(
      core_axis_name="c", subcore_axis_name="s", num_cores=NC,
      num_subcores=NS)
  f = pl.kernel(
      _sc_store,
      out_type=(
          jax.ShapeDtypeStruct((M, D), jnp.float32),
          jax.ShapeDtypeStruct((M,), jnp.int32),
      ),
      mesh=mesh,
      compiler_params=pltpu.CompilerParams(needs_layout_passes=False),
      scratch_types=[
          pltpu.VMEM((B,), jnp.int32),
          pltpu.VMEM((B,), jnp.int32),
          pltpu.VMEM((R,), jnp.int32),
          pltpu.VMEM((R + L,), jnp.int32),
          pltpu.VMEM((R + L,), jnp.int32),
          pltpu.VMEM((R,), jnp.int32),
          pltpu.VMEM((R, D), jnp.float32),
          pltpu.SemaphoreType.DMA,
          pltpu.SemaphoreType.DMA,
          pltpu.SemaphoreType.DMA,
          pltpu.SemaphoreType.DMA,
          pltpu.SemaphoreType.DMA,
      ],
  )
  return f(mem, task_ids, idx, val, new_task_ids)


# speedup vs baseline: 2.1523x; 1.0472x over previous
import functools

import jax
import jax.numpy as jnp
from jax import lax
from jax.experimental import pallas as pl
from jax.experimental.pallas import tpu as pltpu
from jax.experimental.pallas import tpu_sc as plsc

NC = 2
NS = 16
L = 16
UNROLL = 4


def _sc_store(mem_hbm, tid_hbm, idx_hbm, val_hbm, ntid_hbm,
              out_hbm, otid_hbm,
              idx_v, ntid_v, win_v, rows_v, jlist_v, tid_v, mbuf_v,
              isem, nsem, rsem, wsem, gsem, ssem):
  M, D = mem_hbm.shape
  B = idx_hbm.shape[0]
  NW = NC * NS
  R = L * ((M + L * NW - 1) // (L * NW))
  NV = R // L

  w = lax.axis_index("c") * NS + lax.axis_index("s")
  base = jnp.minimum(w * R, M - R)
  lane = lax.iota(jnp.int32, L)

  cp_idx = pltpu.async_copy(idx_hbm, idx_v, isem)
  cp_nt = pltpu.async_copy(ntid_hbm, ntid_v, nsem)
  cp_tid = pltpu.async_copy(tid_hbm.at[pl.ds(base, R)], tid_v, nsem)
  cp_mem = pltpu.async_copy(mem_hbm.at[pl.ds(base, R)], mbuf_v, rsem)

  for i in range(NV):
    win_v[pl.ds(i * L, L)] = jnp.full((L,), -1, jnp.int32)

  cp_idx.wait()

  def phase_a(cc, carry):
    ivs, lasts = [], []
    for u in range(UNROLL):
      iv = idx_v[pl.ds((cc * UNROLL + u) * L, L)]
      ivs.append(iv)
    for u in range(UNROLL):
      _, last = plsc.scan_count(ivs[u])
      lasts.append(last)
    for u in range(UNROLL):
      iv = ivs[u]
      keep = lasts[u] & (iv >= base) & (iv < base + R)
      loc = jnp.where(keep, iv - base, 0)
      plsc.store_scatter(win_v, [loc], (cc * UNROLL + u) * L + lane,
                         mask=keep)
    return carry

  with jax.named_scope("phase_a"):
    lax.fori_loop(0, B // (L * UNROLL), phase_a, 0)

  with jax.named_scope("wait_mem_read"):
    cp_mem.wait()
  cp_out = pltpu.async_copy(mbuf_v, out_hbm.at[pl.ds(base, R)], wsem)

  with jax.named_scope("tid_resolve"):
    cp_nt.wait()
    cp_tid.wait()
    for i in range(NV):
      wv = win_v[pl.ds(i * L, L)]
      have = wv >= 0
      nv = plsc.load_gather(ntid_v, [jnp.where(have, wv, 0)], mask=have)
      tid_v[pl.ds(i * L, L)] = jnp.where(have, nv, tid_v[pl.ds(i * L, L)])
    pltpu.sync_copy(tid_v, otid_hbm.at[pl.ds(base, R)])

  def phase_c(i, cnt):
    wv = win_v[pl.ds(i * L, L)]
    have = wv >= 0
    rowv = base + i * L + lane
    plsc.store_compressed(rows_v.at[pl.ds(cnt, L)], rowv, mask=have)
    plsc.store_compressed(jlist_v.at[pl.ds(cnt, L)], wv, mask=have)
    npc = plsc.all_reduce_population_count(have)
    return cnt + lax.reduce_max(npc, (0,))

  with jax.named_scope("phase_c"):
    cnt = lax.fori_loop(0, NV, phase_c, 0)
  nchunks = (cnt + L - 1) // L

  def pad_lists(c, carry):
    jv = jlist_v[pl.ds(c * L, L)]
    rv = rows_v[pl.ds(c * L, L)]
    valid = (c * L + lane) < cnt
    comp = jnp.where(valid, rv * 8192 + jv, -1)
    pad = plsc.cummax(comp)
    jlist_v[pl.ds(c * L, L)] = jnp.where(
        valid, jv, lax.bitwise_and(pad, 8191))
    rows_v[pl.ds(c * L, L)] = jnp.where(
        valid, rv, lax.shift_right_logical(pad, 13))
    return carry

  lax.fori_loop(jnp.maximum(nchunks - 1, 0), nchunks, pad_lists, 0)

  with jax.named_scope("wait_mem_write"):
    cp_out.wait()

  with jax.named_scope("barrier"):
    plsc.subcore_barrier()

  def fire_gather(c, carry):
    jv = jlist_v[pl.ds(c * L, L)]
    pltpu.async_copy(val_hbm.at[jv], mbuf_v.at[pl.ds(c * L, L)], gsem)
    return carry

  def drain_gather_fire_scatter(c, carry):
    jv = jlist_v[pl.ds(c * L, L)]
    rv = rows_v[pl.ds(c * L, L)]
    pltpu.make_async_copy(
        val_hbm.at[jv], mbuf_v.at[pl.ds(c * L, L)], gsem).wait()
    pltpu.async_copy(mbuf_v.at[pl.ds(c * L, L)], out_hbm.at[rv], ssem)
    return carry

  def drain_scatter(c, carry):
    rv = rows_v[pl.ds(c * L, L)]
    pltpu.make_async_copy(
        mbuf_v.at[pl.ds(c * L, L)], out_hbm.at[rv], ssem).wait()
    return carry

  with jax.named_scope("phase_d"):
    lax.fori_loop(0, nchunks, fire_gather, 0)
    lax.fori_loop(0, nchunks, drain_gather_fire_scatter, 0)
    lax.fori_loop(0, nchunks, drain_scatter, 0)


@jax.jit
def kernel(mem, task_ids, idx, val, new_task_ids):
  M, D = mem.shape
  B = idx.shape[0]
  NW = NC * NS
  R = L * ((M + L * NW - 1) // (L * NW))

  mesh = plsc.VectorSubcoreMesh(
      core_axis_name="c", subcore_axis_name="s", num_cores=NC,
      num_subcores=NS)
  f = pl.kernel(
      _sc_store,
      out_type=(
          jax.ShapeDtypeStruct((M, D), jnp.float32),
          jax.ShapeDtypeStruct((M,), jnp.int32),
      ),
      mesh=mesh,
      compiler_params=pltpu.CompilerParams(needs_layout_passes=False),
      scratch_types=[
          pltpu.VMEM((B,), jnp.int32),
          pltpu.VMEM((B,), jnp.int32),
          pltpu.VMEM((R,), jnp.int32),
          pltpu.VMEM((R + L,), jnp.int32),
          pltpu.VMEM((R + L,), jnp.int32),
          pltpu.VMEM((R,), jnp.int32),
          pltpu.VMEM((R, D), jnp.float32),
          pltpu.SemaphoreType.DMA,
          pltpu.SemaphoreType.DMA,
          pltpu.SemaphoreType.DMA,
          pltpu.SemaphoreType.DMA,
          pltpu.SemaphoreType.DMA,
          pltpu.SemaphoreType.DMA,
      ],
  )
  return f(mem, task_ids, idx, val, new_task_ids)
